# baseline (speedup 1.0000x reference)
import functools

import jax
import jax.numpy as jnp
from jax import lax
from jax.experimental import pallas as pl
from jax.experimental.pallas import tpu as pltpu
from jax.experimental.pallas import tpu_sc as plsc

NUM_CORES = 2
NUM_SUBCORES = 16
NUM_WORKERS = NUM_CORES * NUM_SUBCORES
LANES = 16

SEQS_PER_CHUNK = 2


def _make_lookup(B, S, D):
    assert B % NUM_WORKERS == 0
    seqs_per_worker = B // NUM_WORKERS
    assert seqs_per_worker % SEQS_PER_CHUNK == 0
    chunks = seqs_per_worker // SEQS_PER_CHUNK
    toks = SEQS_PER_CHUNK * S
    assert D == 2 * LANES
    pos_rows128 = S * D // 128

    sub = []
    off = 0
    while off < toks:
        sz = min(128, toks - off)
        sub.append((off, sz))
        off += sz

    mesh = plsc.VectorSubcoreMesh(core_axis_name="c", subcore_axis_name="s")

    @functools.partial(
        pl.kernel,
        mesh=mesh,
        compiler_params=pltpu.CompilerParams(use_tc_tiling_on_sc=True),
        out_type=jax.ShapeDtypeStruct((B, S, D), jnp.float32),
        scratch_types=[
            pltpu.VMEM((toks + LANES,), jnp.int32),
            pltpu.VMEM((toks,), jnp.int32),
            pltpu.VMEM((toks, 128), jnp.float32),
            pltpu.VMEM((pos_rows128, 128), jnp.float32),
            pltpu.VMEM((SEQS_PER_CHUNK, S, D), jnp.float32),
            pltpu.SemaphoreType.DMA,
        ],
    )
    def lookup(table_hbm, idx_hbm, pos_hbm, out_hbm, idx_v, jdx_v, grows_v,
               pos_v, out_v, sem):
        wid = lax.axis_index("s") * NUM_CORES + lax.axis_index("c")
        seq_base = wid * seqs_per_worker

        pltpu.sync_copy(pos_hbm, pos_v)

        def chunk_body(c, carry):
            b0 = pl.multiple_of(seq_base + c * SEQS_PER_CHUNK, SEQS_PER_CHUNK)
            t0 = pl.multiple_of(b0 * S, 8)
            pltpu.sync_copy(idx_hbm.at[pl.ds(t0, toks)], idx_v.at[pl.ds(0, toks)])

            def jdx_body(r, carry2):
                x = idx_v[pl.ds(r * LANES, LANES)]
                jdx_v[pl.ds(r * LANES, LANES)] = lax.shift_right_logical(x, 2)
                return carry2

            lax.fori_loop(0, toks // LANES, jdx_body, 0)

            copies = []
            for (o, sz) in sub:
                copies.append(
                    pltpu.make_async_copy(
                        table_hbm.at[jdx_v.at[pl.ds(o, sz)]],
                        grows_v.at[pl.ds(o, sz)],
                        sem,
                    )
                )
            for cp in copies:
                cp.start()
            for cp in copies:
                cp.wait()

            def add_body(s, carry2):
                prow = s // 4
                pcol = (s % 4) * D
                p0 = pos_v[prow, pl.ds(pcol, LANES)]
                p1 = pos_v[prow, pl.ds(pcol + LANES, LANES)]
                for q in range(SEQS_PER_CHUNK):
                    t = q * S + s
                    goff = (idx_v[pl.ds(t, LANES)][0] & 3) * D
                    out_v[q, s, pl.ds(0, LANES)] = grows_v[t, pl.ds(goff, LANES)] + p0
                    out_v[q, s, pl.ds(LANES, LANES)] = (
                        grows_v[t, pl.ds(goff + LANES, LANES)] + p1
                    )
                return carry2

            lax.fori_loop(0, S, add_body, 0)
            pltpu.sync_copy(out_v, out_hbm.at[pl.ds(b0, SEQS_PER_CHUNK)])
            return carry

        lax.fori_loop(0, chunks, chunk_body, 0)

    return lookup


def kernel(indices, token_table, pos_table):
    B, S = indices.shape
    V, D = token_table.shape
    table128 = token_table.reshape(V * D // 128, 128)
    pos128 = lax.slice(pos_table, (0, 0), (S, D)).reshape(S * D // 128, 128)
    idx_flat = indices.reshape(B * S).astype(jnp.int32)
    lookup = _make_lookup(B, S, D)
    return lookup(table128, idx_flat, pos128)


# speedup vs baseline: 1.6403x; 1.6403x over previous
import functools

import jax
import jax.numpy as jnp
from jax import lax
from jax.experimental import pallas as pl
from jax.experimental.pallas import tpu as pltpu
from jax.experimental.pallas import tpu_sc as plsc

NUM_CORES = 2
NUM_SUBCORES = 16
NUM_WORKERS = NUM_CORES * NUM_SUBCORES
LANES = 16

SEQS_PER_CHUNK = 2


def _make_lookup(B, S, D):
    assert B % NUM_WORKERS == 0
    seqs_per_worker = B // NUM_WORKERS
    assert seqs_per_worker % (2 * SEQS_PER_CHUNK) == 0
    chunks = seqs_per_worker // SEQS_PER_CHUNK
    toks = SEQS_PER_CHUNK * S
    assert D == 2 * LANES

    sub = []
    for q in range(SEQS_PER_CHUNK):
        off = 0
        while off < S:
            sz = min(128, S - off)
            sub.append((q, off, sz))
            off += sz

    mesh = plsc.VectorSubcoreMesh(core_axis_name="c", subcore_axis_name="s")

    @functools.partial(
        pl.kernel,
        mesh=mesh,
        compiler_params=pltpu.CompilerParams(use_tc_tiling_on_sc=False),
        out_type=jax.ShapeDtypeStruct((B, S, D), jnp.float32),
        scratch_types=[
            pltpu.VMEM((SEQS_PER_CHUNK, S), jnp.int32),
            pltpu.VMEM((SEQS_PER_CHUNK, S), jnp.int32),
            pltpu.VMEM((SEQS_PER_CHUNK, S, D), jnp.float32),
            pltpu.VMEM((SEQS_PER_CHUNK, S, D), jnp.float32),
            pltpu.VMEM((S, D), jnp.float32),
            pltpu.SemaphoreType.DMA,
            pltpu.SemaphoreType.DMA,
            pltpu.SemaphoreType.DMA,
            pltpu.SemaphoreType.DMA,
        ],
    )
    def lookup(table_hbm, idx_hbm, pos_hbm, out_hbm,
               idx0_v, idx1_v, rows0_v, rows1_v, pos_v,
               gsem0, gsem1, osem0, osem1):
        wid = lax.axis_index("s") * NUM_CORES + lax.axis_index("c")
        seq_base = wid * seqs_per_worker

        idx_v = (idx0_v, idx1_v)
        rows_v = (rows0_v, rows1_v)
        gsem = (gsem0, gsem1)
        osem = (osem0, osem1)

        pltpu.sync_copy(pos_hbm, pos_v)

        def gather_copies(c, buf):
            b0 = pl.multiple_of(seq_base + c * SEQS_PER_CHUNK, SEQS_PER_CHUNK)
            cps = []
            for (q, r, sz) in sub:
                cps.append(
                    pltpu.make_async_copy(
                        table_hbm.at[idx_v[buf].at[q, pl.ds(r, sz)]],
                        rows_v[buf].at[q, pl.ds(r, sz)],
                        gsem[buf],
                    )
                )
            return b0, cps

        def start_chunk(c, buf):
            b0 = pl.multiple_of(seq_base + c * SEQS_PER_CHUNK, SEQS_PER_CHUNK)
            pltpu.sync_copy(idx_hbm.at[pl.ds(b0, SEQS_PER_CHUNK)], idx_v[buf])
            _, cps = gather_copies(c, buf)
            for cp in cps:
                cp.start()

        def out_copy(c, buf):
            b0 = pl.multiple_of(seq_base + c * SEQS_PER_CHUNK, SEQS_PER_CHUNK)
            return pltpu.make_async_copy(
                rows_v[buf], out_hbm.at[pl.ds(b0, SEQS_PER_CHUNK)], osem[buf]
            )

        def add_pos(buf):
            def add_body(s, carry2):
                p0 = pos_v[s, pl.ds(0, LANES)]
                p1 = pos_v[s, pl.ds(LANES, LANES)]
                for q in range(SEQS_PER_CHUNK):
                    rows_v[buf][q, s, pl.ds(0, LANES)] += p0
                    rows_v[buf][q, s, pl.ds(LANES, LANES)] += p1
                return carry2

            lax.fori_loop(0, S, add_body, 0)

        start_chunk(0, 0)

        def step(i, carry):
            for buf in (0, 1):
                c = i * 2 + buf
                _, cps = gather_copies(c, buf)
                for cp in cps:
                    cp.wait()
                @pl.when(c >= 1)
                def _():
                    out_copy(c - 1, 1 - buf).wait()

                @pl.when(c + 1 < chunks)
                def _():
                    start_chunk(c + 1, 1 - buf)

                add_pos(buf)
                out_copy(c, buf).start()
            return carry

        lax.fori_loop(0, chunks // 2, step, 0)
        out_copy(chunks - 1, 1).wait()

    return lookup


def kernel(indices, token_table, pos_table):
    B, S = indices.shape
    V, D = token_table.shape
    pos_rows = lax.slice(pos_table, (0, 0), (S, D))
    lookup = _make_lookup(B, S, D)
    return lookup(token_table, indices.astype(jnp.int32), pos_rows)
